# baseline (speedup 1.0000x reference)
import jax
import jax.numpy as jnp
from jax.experimental import pallas as pl

_FEAT_DIM = 512
_CATE_NUM = 1000
_SHOTS_TOTAL = 32
_FEAT_NUM = 256
_ROWS = _CATE_NUM * _SHOTS_TOTAL

_BLK_CLS = 40
_NSTEP = _CATE_NUM // _BLK_CLS


def _tc_small_body(idx_ref, res_full_ref, cw_ref, ncw_ref, rexp_ref):
    col = jax.lax.broadcasted_iota(jnp.int32, (_FEAT_NUM, _FEAT_DIM), 1)
    s = (idx_ref[...] == col).astype(jnp.float32)
    rexp_ref[...] = jnp.dot(res_full_ref[...], s,
                            preferred_element_type=jnp.float32)
    ncw_ref[...] = cw_ref[...] + jax.lax.dot_general(
        s, res_full_ref[...], (((0,), (1,)), ((), ())),
        preferred_element_type=jnp.float32)


def _tc_ck_body(ck_ref, rexp_ref, nck_ref):
    rep = jnp.broadcast_to(rexp_ref[...][:, None, :],
                           (_BLK_CLS, _SHOTS_TOTAL, _FEAT_DIM))
    nck_ref[...] = ck_ref[...] + rep.reshape(_BLK_CLS * _SHOTS_TOTAL, _FEAT_DIM)


def _tc_cv_body(cv_ref, vw_ref, ncv_ref):
    ncv_ref[...] = cv_ref[...] * vw_ref[...]


def kernel(cache_keys, clip_weights, cache_values, res, value_weights, indices):
    idx = indices.astype(jnp.int32).reshape(_FEAT_NUM, 1)
    ncw, rexp = pl.pallas_call(
        _tc_small_body,
        in_specs=[
            pl.BlockSpec((_FEAT_NUM, 1), lambda: (0, 0)),
            pl.BlockSpec((_CATE_NUM, _FEAT_NUM), lambda: (0, 0)),
            pl.BlockSpec((_FEAT_DIM, _CATE_NUM), lambda: (0, 0)),
        ],
        out_specs=[
            pl.BlockSpec((_FEAT_DIM, _CATE_NUM), lambda: (0, 0)),
            pl.BlockSpec((_CATE_NUM, _FEAT_DIM), lambda: (0, 0)),
        ],
        out_shape=[
            jax.ShapeDtypeStruct((_FEAT_DIM, _CATE_NUM), jnp.float32),
            jax.ShapeDtypeStruct((_CATE_NUM, _FEAT_DIM), jnp.float32),
        ],
    )(idx, res, clip_weights)

    nck = pl.pallas_call(
        _tc_ck_body,
        grid=(_NSTEP,),
        in_specs=[
            pl.BlockSpec((_BLK_CLS * _SHOTS_TOTAL, _FEAT_DIM), lambda i: (i, 0)),
            pl.BlockSpec((_BLK_CLS, _FEAT_DIM), lambda i: (i, 0)),
        ],
        out_specs=pl.BlockSpec((_BLK_CLS * _SHOTS_TOTAL, _FEAT_DIM),
                               lambda i: (i, 0)),
        out_shape=jax.ShapeDtypeStruct((_ROWS, _FEAT_DIM), jnp.float32),
    )(cache_keys, rexp)

    halves = []
    hc = _CATE_NUM // 2
    hstep = _NSTEP // 2
    for h in range(2):
        cv3 = jax.lax.slice_in_dim(cache_values, h * (_ROWS // 2), (h + 1) * (_ROWS // 2),
                                   axis=0).reshape(hc, _SHOTS_TOTAL, _CATE_NUM)
        vw3 = jax.lax.slice_in_dim(value_weights, h * (_ROWS // 2), (h + 1) * (_ROWS // 2),
                                   axis=0).reshape(hc, _SHOTS_TOTAL, 1)
        ncv3 = pl.pallas_call(
            _tc_cv_body,
            grid=(hc // 50,),
            in_specs=[
                pl.BlockSpec((50, _SHOTS_TOTAL, _CATE_NUM), lambda i: (i, 0, 0)),
                pl.BlockSpec((50, _SHOTS_TOTAL, 1), lambda i: (i, 0, 0)),
            ],
            out_specs=pl.BlockSpec((50, _SHOTS_TOTAL, _CATE_NUM),
                                   lambda i: (i, 0, 0)),
            out_shape=jax.ShapeDtypeStruct((hc, _SHOTS_TOTAL, _CATE_NUM),
                                           jnp.float32),
        )(cv3, vw3)
        halves.append(ncv3.reshape(_ROWS // 2, _CATE_NUM))
    ncv = jnp.concatenate(halves, axis=0)
    return (nck, ncw, ncv)
